# Initial kernel scaffold; baseline (speedup 1.0000x reference)
import jax
import jax.numpy as jnp
from jax.experimental import pallas as pl


def kernel(atom_output, bond_output, original_f_atoms, original_f_bonds, a2a, a2b, b2a, b2revb, a_scope, features_batch, bond_in_atom_out, W1_bfa, b1_bfa, W2_bfa, b2_bfa, g_bfa, be_bfa, W1_bfb, b1_bfb, W2_bfb, b2_bfb, g_bfb, be_bfb, Wf1, bf1, Wf2, bf2):
    raise NotImplementedError("write your pallas kernel here")


# speedup vs baseline: 5.1443x; 5.1443x over previous
import functools

import jax
import jax.numpy as jnp
from jax import lax
from jax.experimental import pallas as pl
from jax.experimental.pallas import tpu as pltpu
from jax.experimental.pallas import tpu_sc as plsc

F32 = jnp.float32
BF16 = jnp.bfloat16
I32 = jnp.int32

_NC = 2
_NS = 16
_NW = _NC * _NS


def _build_ac_table(atom_output, bond_output, a2a_flat, a2b_flat):
    A, H = atom_output.shape
    NB = a2a_flat.shape[0] // A
    G = 4
    NG = A // G
    ITERS = (NG + _NW - 1) // _NW
    mesh = plsc.VectorSubcoreMesh(core_axis_name="c", subcore_axis_name="s")

    @functools.partial(
        pl.kernel, mesh=mesh,
        out_type=jax.ShapeDtypeStruct((A, 2 * H), F32),
        scratch_types=[
            pltpu.VMEM((G * NB,), I32),
            pltpu.VMEM((G * NB,), I32),
            pltpu.VMEM((G * NB, H), F32),
            pltpu.VMEM((G * NB, H), F32),
            pltpu.VMEM((G, H), F32),
            pltpu.VMEM((G, 2 * H), F32),
            pltpu.SemaphoreType.DMA,
            pltpu.SemaphoreType.DMA,
        ],
    )
    def k(atom_hbm, bond_hbm, a2a_hbm, a2b_hbm, out_hbm,
          idx_a, idx_b, nbr_a, nbr_b, selfr, outr, sem_a, sem_b):
        wid = lax.axis_index("s") * _NC + lax.axis_index("c")

        def body(it, carry):
            gi = wid + it * _NW

            @pl.when(gi < NG)
            def _():
                a0 = gi * G
                pltpu.sync_copy(a2a_hbm.at[pl.ds(a0 * NB, G * NB)], idx_a)
                pltpu.sync_copy(a2b_hbm.at[pl.ds(a0 * NB, G * NB)], idx_b)
                cp_a = pltpu.async_copy(atom_hbm.at[idx_a], nbr_a, sem_a)
                cp_b = pltpu.async_copy(bond_hbm.at[idx_b], nbr_b, sem_b)
                pltpu.sync_copy(atom_hbm.at[pl.ds(a0, G)], selfr)
                cp_a.wait()
                cp_b.wait()

                def col(c, carry2):
                    s = pl.ds(c * 16, 16)
                    for a in range(G):
                        acc = selfr[a, s]
                        acc2 = nbr_b[a * NB, s]
                        for j in range(NB):
                            acc = acc + nbr_a[a * NB + j, s]
                        for j in range(1, NB):
                            acc2 = acc2 + nbr_b[a * NB + j, s]
                        outr[a, s] = acc
                        outr[a, pl.ds(H + c * 16, 16)] = acc2
                    return carry2

                lax.fori_loop(0, H // 16, col, 0)
                pltpu.sync_copy(outr, out_hbm.at[pl.ds(a0, G)])
            return carry

        lax.fori_loop(0, ITERS, body, 0)

    return k(atom_output, bond_output, a2a_flat, a2b_flat)


def _build_aggr(ac_table, atom_output, bond_output, b2a, b2revb):
    A, H2 = ac_table.shape
    H = H2 // 2
    E = b2a.shape[0]
    PER_W = E // _NW
    C = 40
    NCH = PER_W // C
    mesh = plsc.VectorSubcoreMesh(core_axis_name="c", subcore_axis_name="s")

    @functools.partial(
        pl.kernel, mesh=mesh,
        out_type=jax.ShapeDtypeStruct((E, 2 * H), F32),
        scratch_types=[
            pltpu.VMEM((C,), I32),
            pltpu.VMEM((C,), I32),
            pltpu.VMEM((C,), I32),
            pltpu.VMEM((C, 2 * H), F32),
            pltpu.VMEM((C, H), F32),
            pltpu.VMEM((C, H), F32),
            pltpu.SemaphoreType.DMA,
            pltpu.SemaphoreType.DMA,
            pltpu.SemaphoreType.DMA,
        ],
    )
    def k(ac_hbm, atom_hbm, bond_hbm, b2a_hbm, brev_hbm, out_hbm,
          ba, brev, reva, acr, br, dr, s0, s1, s2):
        wid = lax.axis_index("s") * _NC + lax.axis_index("c")
        base = wid * PER_W

        def body(i, carry):
            e0 = base + i * C
            pltpu.sync_copy(b2a_hbm.at[pl.ds(e0, C)], ba)
            pltpu.sync_copy(brev_hbm.at[pl.ds(e0, C)], brev)
            pltpu.sync_copy(b2a_hbm.at[brev], reva)
            cp0 = pltpu.async_copy(ac_hbm.at[ba], acr, s0)
            cp1 = pltpu.async_copy(atom_hbm.at[reva], br, s1)
            cp2 = pltpu.async_copy(bond_hbm.at[brev], dr, s2)
            cp0.wait()
            cp1.wait()
            cp2.wait()

            def row(r, carry2):
                def colfn(c, carry3):
                    s = pl.ds(c * 16, 16)
                    sh = pl.ds(H + c * 16, 16)
                    acr[r, s] = acr[r, s] - br[r, s]
                    acr[r, sh] = acr[r, sh] - dr[r, s]
                    return carry3
                return lax.fori_loop(0, H // 16, colfn, carry2)

            lax.fori_loop(0, C, row, 0)
            pltpu.sync_copy(acr, out_hbm.at[pl.ds(e0, C)])
            return carry

        lax.fori_loop(0, NCH, body, 0)

    return k(ac_table, atom_output, bond_output, b2a, b2revb)


def _ffn_body(f_ref, ag_ref,
              w1a_ref, b1a_ref, w2a_ref, b2a_ref, ga_ref, bea_ref,
              w1b_ref, b1b_ref, w2b_ref, b2b_ref, gb_ref, beb_ref,
              oa_ref, ob_ref):
    H = f_ref.shape[1]
    f = f_ref[...]
    ag = ag_ref[...]
    branches = (
        (ag[:, :H], w1a_ref, b1a_ref, w2a_ref, b2a_ref, ga_ref, bea_ref, oa_ref),
        (ag[:, H:], w1b_ref, b1b_ref, w2b_ref, b2b_ref, gb_ref, beb_ref, ob_ref),
    )
    for aggr, w1r, b1r, w2r, b2r, gr, ber, outr in branches:
        x = jnp.concatenate([f, aggr], axis=1).astype(BF16)
        h = jnp.dot(x, w1r[...], preferred_element_type=F32) + b1r[...]
        h = jnp.maximum(h, 0.0).astype(BF16)
        y = jnp.dot(h, w2r[...], preferred_element_type=F32) + b2r[...]
        m = jnp.mean(y, axis=1, keepdims=True)
        yc = y - m
        v = jnp.mean(yc * yc, axis=1, keepdims=True)
        outr[...] = yc * lax.rsqrt(v + 1e-5) * gr[...] + ber[...]


def _run_ffn(f_bonds, aggr, W1a, b1a, W2a, b2a_, ga, bea,
             W1b, b1b, W2b, b2b_, gb, beb):
    E, H = f_bonds.shape
    D = W1a.shape[1]
    BE = 640
    grid = (E // BE,)
    row_spec = lambda w: pl.BlockSpec((BE, w), lambda i: (i, 0))
    full_spec = lambda r, c: pl.BlockSpec((r, c), lambda i: (0, 0))
    out = pl.pallas_call(
        _ffn_body,
        grid=grid,
        in_specs=[
            row_spec(H), row_spec(2 * H),
            full_spec(2 * H, D), full_spec(1, D), full_spec(D, H),
            full_spec(1, H), full_spec(1, H), full_spec(1, H),
            full_spec(2 * H, D), full_spec(1, D), full_spec(D, H),
            full_spec(1, H), full_spec(1, H), full_spec(1, H),
        ],
        out_specs=[row_spec(H), row_spec(H)],
        out_shape=[jax.ShapeDtypeStruct((E, H), F32),
                   jax.ShapeDtypeStruct((E, H), F32)],
    )(f_bonds, aggr,
      W1a, b1a, W2a, b2a_, ga, bea,
      W1b, b1b, W2b, b2b_, gb, beb)
    return out


def _readout_body(bia_ref, sizes_ref, feats_ref, w1m_ref, w1f_ref, b1_ref,
                  w2_ref, b2_ref, out_ref, *, n_mols, mol_sz):
    x = bia_ref[...]
    H = x.shape[1]
    xr = x.reshape(n_mols, mol_sz, H)
    sizes = sizes_ref[...]
    offs = lax.broadcasted_iota(I32, (n_mols, mol_sz), 1).astype(F32)
    mask = (offs < sizes).astype(F32)
    mv = (xr * mask[:, :, None]).sum(axis=1) / sizes
    h = jnp.dot(mv, w1m_ref[...], preferred_element_type=F32)
    h = h + jnp.dot(feats_ref[...], w1f_ref[...], preferred_element_type=F32)
    h = jnp.maximum(h + b1_ref[...], 0.0)
    y = jnp.dot(h, w2_ref[...], preferred_element_type=F32) + b2_ref[...]
    out_ref[...] = jax.nn.sigmoid(y) * 0.5


def _run_readout(bond_in_atom_out, sizes_f, features, Wf1, bf1, Wf2, bf2):
    NA, H = bond_in_atom_out.shape
    M, FT = features.shape
    mol_sz = NA // M
    FH = Wf1.shape[1]
    T = Wf2.shape[1]
    body = functools.partial(_readout_body, n_mols=M, mol_sz=mol_sz)
    return pl.pallas_call(
        body,
        out_shape=jax.ShapeDtypeStruct((M, T), F32),
    )(bond_in_atom_out, sizes_f, features,
      Wf1[:H], Wf1[H:], bf1.reshape(1, FH), Wf2, bf2.reshape(1, T))


def kernel(atom_output, bond_output, original_f_atoms, original_f_bonds,
           a2a, a2b, b2a, b2revb, a_scope, features_batch, bond_in_atom_out,
           W1_bfa, b1_bfa, W2_bfa, b2_bfa, g_bfa, be_bfa,
           W1_bfb, b1_bfb, W2_bfb, b2_bfb, g_bfb, be_bfb,
           Wf1, bf1, Wf2, bf2):
    H = atom_output.shape[1]
    a2a_flat = a2a.astype(I32).reshape(-1)
    a2b_flat = a2b.astype(I32).reshape(-1)
    b2a_i = b2a.astype(I32)
    brev_i = b2revb.astype(I32)

    ac = _build_ac_table(atom_output, bond_output, a2a_flat, a2b_flat)
    aggr = _build_aggr(ac, atom_output, bond_output, b2a_i, brev_i)

    out_a, out_b = _run_ffn(
        original_f_bonds, aggr,
        W1_bfa.astype(BF16), b1_bfa.reshape(1, -1), W2_bfa.astype(BF16),
        b2_bfa.reshape(1, -1), g_bfa.reshape(1, -1), be_bfa.reshape(1, -1),
        W1_bfb.astype(BF16), b1_bfb.reshape(1, -1), W2_bfb.astype(BF16),
        b2_bfb.reshape(1, -1), g_bfb.reshape(1, -1), be_bfb.reshape(1, -1))

    sizes_f = a_scope[:, 1].astype(F32).reshape(-1, 1)
    output = _run_readout(bond_in_atom_out, sizes_f, features_batch,
                          Wf1, bf1, Wf2, bf2)
    return output, out_a, out_b


# speedup vs baseline: 6.6915x; 1.3007x over previous
import functools

import jax
import jax.numpy as jnp
from jax import lax
from jax.experimental import pallas as pl
from jax.experimental.pallas import tpu as pltpu
from jax.experimental.pallas import tpu_sc as plsc

F32 = jnp.float32
BF16 = jnp.bfloat16
I32 = jnp.int32

_NC = 2
_NS = 16
_NW = _NC * _NS


def _build_agg_tables(atom_output, bond_output, a2a_flat, a2b_flat):
    A, H = atom_output.shape
    NB = a2a_flat.shape[0] // A
    G = 4
    NG = A // G
    ITERS = (NG + _NW - 1) // _NW
    mesh = plsc.VectorSubcoreMesh(core_axis_name="c", subcore_axis_name="s")

    @functools.partial(
        pl.kernel, mesh=mesh,
        out_type=[jax.ShapeDtypeStruct((A, H), F32),
                  jax.ShapeDtypeStruct((A, H), F32)],
        scratch_types=[
            pltpu.VMEM((G * NB,), I32),
            pltpu.VMEM((G * NB,), I32),
            pltpu.VMEM((G * NB, H), F32),
            pltpu.VMEM((G * NB, H), F32),
            pltpu.VMEM((G, H), F32),
            pltpu.VMEM((G, H), F32),
            pltpu.VMEM((G, H), F32),
            pltpu.SemaphoreType.DMA,
            pltpu.SemaphoreType.DMA,
        ],
    )
    def k(atom_hbm, bond_hbm, a2a_hbm, a2b_hbm, outa_hbm, outb_hbm,
          idx_a, idx_b, nbr_a, nbr_b, selfr, outra, outrb, sem_a, sem_b):
        wid = lax.axis_index("s") * _NC + lax.axis_index("c")

        def body(it, carry):
            gi = wid + it * _NW

            @pl.when(gi < NG)
            def _():
                a0 = gi * G
                pltpu.sync_copy(a2a_hbm.at[pl.ds(a0 * NB, G * NB)], idx_a)
                pltpu.sync_copy(a2b_hbm.at[pl.ds(a0 * NB, G * NB)], idx_b)
                cp_a = pltpu.async_copy(atom_hbm.at[idx_a], nbr_a, sem_a)
                cp_b = pltpu.async_copy(bond_hbm.at[idx_b], nbr_b, sem_b)
                pltpu.sync_copy(atom_hbm.at[pl.ds(a0, G)], selfr)
                cp_a.wait()
                cp_b.wait()

                def col(c, carry2):
                    s = pl.ds(c * 16, 16)
                    for a in range(G):
                        acc = selfr[a, s]
                        acc2 = nbr_b[a * NB, s]
                        for j in range(NB):
                            acc = acc + nbr_a[a * NB + j, s]
                        for j in range(1, NB):
                            acc2 = acc2 + nbr_b[a * NB + j, s]
                        outra[a, s] = acc
                        outrb[a, s] = acc2
                    return carry2

                lax.fori_loop(0, H // 16, col, 0)
                pltpu.sync_copy(outra, outa_hbm.at[pl.ds(a0, G)])
                pltpu.sync_copy(outrb, outb_hbm.at[pl.ds(a0, G)])
            return carry

        lax.fori_loop(0, ITERS, body, 0)

    return k(atom_output, bond_output, a2a_flat, a2b_flat)


def _build_aggr(agg_atom, agg_bond, atom_output, bond_output, b2a, b2revb):
    A, H = agg_atom.shape
    E = b2a.shape[0]
    PER_W = E // _NW
    C = 32
    NCH = PER_W // C
    TAIL = PER_W - NCH * C
    EG = 128
    NEG = (PER_W + EG - 1) // EG
    mesh = plsc.VectorSubcoreMesh(core_axis_name="c", subcore_axis_name="s")

    @functools.partial(
        pl.kernel, mesh=mesh,
        out_type=[jax.ShapeDtypeStruct((E, H), F32),
                  jax.ShapeDtypeStruct((E, H), F32)],
        scratch_types=[
            pltpu.VMEM((PER_W,), I32),
            pltpu.VMEM((PER_W,), I32),
            pltpu.VMEM((PER_W,), I32),
            pltpu.VMEM((C, H), F32),
            pltpu.VMEM((C, H), F32),
            pltpu.VMEM((C, H), F32),
            pltpu.VMEM((C, H), F32),
            pltpu.VMEM((C, H), F32),
            pltpu.VMEM((C, H), F32),
            pltpu.SemaphoreType.DMA,
            pltpu.SemaphoreType.DMA,
            pltpu.SemaphoreType.DMA,
            pltpu.SemaphoreType.DMA,
            pltpu.SemaphoreType.DMA,
            pltpu.SemaphoreType.DMA,
            pltpu.SemaphoreType.DMA,
        ],
    )
    def k(ga_hbm, gb_hbm, atom_hbm, bond_hbm, b2a_hbm, brev_hbm,
          outa_hbm, outb_hbm,
          ba_all, brev_all, reva_all, x0, y0, x1, y1, z0, z1,
          sx0, sy0, sx1, sy1, sw0, sw1, se):
        wid = lax.axis_index("s") * _NC + lax.axis_index("c")
        base = wid * PER_W

        pltpu.sync_copy(b2a_hbm.at[pl.ds(base, PER_W)], ba_all)
        pltpu.sync_copy(brev_hbm.at[pl.ds(base, PER_W)], brev_all)
        BATCH = 10
        for j0 in range(0, NEG, BATCH):
            js = range(j0, min(j0 + BATCH, NEG))
            cps = []
            for j in js:
                n = min(EG, PER_W - j * EG)
                cps.append(pltpu.async_copy(
                    b2a_hbm.at[brev_all.at[pl.ds(j * EG, n)]],
                    reva_all.at[pl.ds(j * EG, n)], se))
            for cp in cps:
                cp.wait()

        def gathers_a(c):
            return (
                pltpu.make_async_copy(
                    ga_hbm.at[ba_all.at[pl.ds(c * C, C)]], x0, sx0),
                pltpu.make_async_copy(
                    atom_hbm.at[reva_all.at[pl.ds(c * C, C)]], y0, sy0),
            )

        def gathers_b(c):
            return (
                pltpu.make_async_copy(
                    gb_hbm.at[ba_all.at[pl.ds(c * C, C)]], x1, sx1),
                pltpu.make_async_copy(
                    bond_hbm.at[brev_all.at[pl.ds(c * C, C)]], y1, sy1),
            )

        def issue(cps):
            for cp in cps:
                cp.start()

        def drain(hbm, dst, sem):
            pltpu.make_async_copy(hbm.at[pl.ds(0, C)], dst, sem).wait()

        def subtract(xr, yr, zr):
            def row(r, carry):
                def colfn(cc, carry2):
                    s = pl.ds(cc * 16, 16)
                    zr[r, s] = xr[r, s] - yr[r, s]
                    return carry2
                return lax.fori_loop(0, H // 16, colfn, carry)
            lax.fori_loop(0, C, row, 0)

        issue(gathers_a(0))
        issue(gathers_b(0))

        def body(c, carry):
            ca_x, ca_y = gathers_a(c)
            ca_x.wait()
            ca_y.wait()

            @pl.when(c > 0)
            def _():
                drain(outa_hbm, z0, sw0)
            subtract(x0, y0, z0)

            @pl.when(c + 1 < NCH)
            def _():
                issue(gathers_a(c + 1))
            pltpu.async_copy(z0, outa_hbm.at[pl.ds(base + c * C, C)], sw0)

            cb_x, cb_y = gathers_b(c)
            cb_x.wait()
            cb_y.wait()

            @pl.when(c > 0)
            def _():
                drain(outb_hbm, z1, sw1)
            subtract(x1, y1, z1)

            @pl.when(c + 1 < NCH)
            def _():
                issue(gathers_b(c + 1))
            pltpu.async_copy(z1, outb_hbm.at[pl.ds(base + c * C, C)], sw1)
            return carry

        lax.fori_loop(0, NCH, body, 0)

        drain(outa_hbm, z0, sw0)
        drain(outb_hbm, z1, sw1)

        if TAIL:
            t0 = NCH * C
            pltpu.async_copy(
                ga_hbm.at[ba_all.at[pl.ds(t0, TAIL)]],
                x0.at[pl.ds(0, TAIL)], sx0).wait()
            pltpu.async_copy(
                atom_hbm.at[reva_all.at[pl.ds(t0, TAIL)]],
                y0.at[pl.ds(0, TAIL)], sy0).wait()
            pltpu.async_copy(
                gb_hbm.at[ba_all.at[pl.ds(t0, TAIL)]],
                x1.at[pl.ds(0, TAIL)], sx1).wait()
            pltpu.async_copy(
                bond_hbm.at[brev_all.at[pl.ds(t0, TAIL)]],
                y1.at[pl.ds(0, TAIL)], sy1).wait()

            def trow(r, carry):
                def tcol(cc, carry2):
                    s = pl.ds(cc * 16, 16)
                    z0[r, s] = x0[r, s] - y0[r, s]
                    z1[r, s] = x1[r, s] - y1[r, s]
                    return carry2
                return lax.fori_loop(0, H // 16, tcol, carry)
            lax.fori_loop(0, TAIL, trow, 0)
            pltpu.sync_copy(z0.at[pl.ds(0, TAIL)],
                            outa_hbm.at[pl.ds(base + t0, TAIL)])
            pltpu.sync_copy(z1.at[pl.ds(0, TAIL)],
                            outb_hbm.at[pl.ds(base + t0, TAIL)])

    return k(agg_atom, agg_bond, atom_output, bond_output, b2a, b2revb)


def _ffn_body(f_ref, aga_ref, agb_ref,
              w1a_ref, b1a_ref, w2a_ref, b2a_ref, ga_ref, bea_ref,
              w1b_ref, b1b_ref, w2b_ref, b2b_ref, gb_ref, beb_ref,
              oa_ref, ob_ref):
    f = f_ref[...]
    branches = (
        (aga_ref, w1a_ref, b1a_ref, w2a_ref, b2a_ref, ga_ref, bea_ref, oa_ref),
        (agb_ref, w1b_ref, b1b_ref, w2b_ref, b2b_ref, gb_ref, beb_ref, ob_ref),
    )
    for agr, w1r, b1r, w2r, b2r, gr, ber, outr in branches:
        x = jnp.concatenate([f, agr[...]], axis=1).astype(BF16)
        h = jnp.dot(x, w1r[...], preferred_element_type=F32) + b1r[...]
        h = jnp.maximum(h, 0.0).astype(BF16)
        y = jnp.dot(h, w2r[...], preferred_element_type=F32) + b2r[...]
        m = jnp.mean(y, axis=1, keepdims=True)
        yc = y - m
        v = jnp.mean(yc * yc, axis=1, keepdims=True)
        outr[...] = yc * lax.rsqrt(v + 1e-5) * gr[...] + ber[...]


def _run_ffn(f_bonds, aggr_a, aggr_b, W1a, b1a, W2a, b2a_, ga, bea,
             W1b, b1b, W2b, b2b_, gb, beb):
    E, H = f_bonds.shape
    D = W1a.shape[1]
    BE = 640
    grid = (E // BE,)
    row_spec = lambda w: pl.BlockSpec((BE, w), lambda i: (i, 0))
    full_spec = lambda r, c: pl.BlockSpec((r, c), lambda i: (0, 0))
    out = pl.pallas_call(
        _ffn_body,
        grid=grid,
        in_specs=[
            row_spec(H), row_spec(H), row_spec(H),
            full_spec(2 * H, D), full_spec(1, D), full_spec(D, H),
            full_spec(1, H), full_spec(1, H), full_spec(1, H),
            full_spec(2 * H, D), full_spec(1, D), full_spec(D, H),
            full_spec(1, H), full_spec(1, H), full_spec(1, H),
        ],
        out_specs=[row_spec(H), row_spec(H)],
        out_shape=[jax.ShapeDtypeStruct((E, H), F32),
                   jax.ShapeDtypeStruct((E, H), F32)],
    )(f_bonds, aggr_a, aggr_b,
      W1a, b1a, W2a, b2a_, ga, bea,
      W1b, b1b, W2b, b2b_, gb, beb)
    return out


def _readout_body(bia_ref, sizes_ref, feats_ref, w1m_ref, w1f_ref, b1_ref,
                  w2_ref, b2_ref, out_ref, *, n_mols, mol_sz):
    x = bia_ref[...]
    H = x.shape[1]
    xr = x.reshape(n_mols, mol_sz, H)
    sizes = sizes_ref[...]
    offs = lax.broadcasted_iota(I32, (n_mols, mol_sz), 1).astype(F32)
    mask = (offs < sizes).astype(F32)
    mv = (xr * mask[:, :, None]).sum(axis=1) / sizes
    h = jnp.dot(mv, w1m_ref[...], preferred_element_type=F32)
    h = h + jnp.dot(feats_ref[...], w1f_ref[...], preferred_element_type=F32)
    h = jnp.maximum(h + b1_ref[...], 0.0)
    y = jnp.dot(h, w2_ref[...], preferred_element_type=F32) + b2_ref[...]
    out_ref[...] = jax.nn.sigmoid(y) * 0.5


def _run_readout(bond_in_atom_out, sizes_f, features, Wf1, bf1, Wf2, bf2):
    NA, H = bond_in_atom_out.shape
    M, FT = features.shape
    mol_sz = NA // M
    FH = Wf1.shape[1]
    T = Wf2.shape[1]
    body = functools.partial(_readout_body, n_mols=M, mol_sz=mol_sz)
    return pl.pallas_call(
        body,
        out_shape=jax.ShapeDtypeStruct((M, T), F32),
    )(bond_in_atom_out, sizes_f, features,
      Wf1[:H], Wf1[H:], bf1.reshape(1, FH), Wf2, bf2.reshape(1, T))


def kernel(atom_output, bond_output, original_f_atoms, original_f_bonds,
           a2a, a2b, b2a, b2revb, a_scope, features_batch, bond_in_atom_out,
           W1_bfa, b1_bfa, W2_bfa, b2_bfa, g_bfa, be_bfa,
           W1_bfb, b1_bfb, W2_bfb, b2_bfb, g_bfb, be_bfb,
           Wf1, bf1, Wf2, bf2):
    a2a_flat = a2a.astype(I32).reshape(-1)
    a2b_flat = a2b.astype(I32).reshape(-1)
    b2a_i = b2a.astype(I32)
    brev_i = b2revb.astype(I32)

    agg_atom, agg_bond = _build_agg_tables(
        atom_output, bond_output, a2a_flat, a2b_flat)
    aggr_a, aggr_b = _build_aggr(
        agg_atom, agg_bond, atom_output, bond_output, b2a_i, brev_i)

    out_a, out_b = _run_ffn(
        original_f_bonds, aggr_a, aggr_b,
        W1_bfa.astype(BF16), b1_bfa.reshape(1, -1), W2_bfa.astype(BF16),
        b2_bfa.reshape(1, -1), g_bfa.reshape(1, -1), be_bfa.reshape(1, -1),
        W1_bfb.astype(BF16), b1_bfb.reshape(1, -1), W2_bfb.astype(BF16),
        b2_bfb.reshape(1, -1), g_bfb.reshape(1, -1), be_bfb.reshape(1, -1))

    sizes_f = a_scope[:, 1].astype(F32).reshape(-1, 1)
    output = _run_readout(bond_in_atom_out, sizes_f, features_batch,
                          Wf1, bf1, Wf2, bf2)
    return output, out_a, out_b


# speedup vs baseline: 7.9391x; 1.1865x over previous
import functools

import jax
import jax.numpy as jnp
from jax import lax
from jax.experimental import pallas as pl
from jax.experimental.pallas import tpu as pltpu
from jax.experimental.pallas import tpu_sc as plsc

F32 = jnp.float32
BF16 = jnp.bfloat16
I32 = jnp.int32

_NC = 2
_NS = 16
_NW = _NC * _NS


def _build_agg_tables(atom_output, bond_output, a2a_flat, a2b_flat):
    A, H = atom_output.shape
    NB = 16
    G = 4
    AP = 320
    NCH = AP // G
    mesh = plsc.VectorSubcoreMesh(core_axis_name="c", subcore_axis_name="s")

    @functools.partial(
        pl.kernel, mesh=mesh,
        out_type=[jax.ShapeDtypeStruct((A, H), F32),
                  jax.ShapeDtypeStruct((A, H), F32)],
        scratch_types=[
            pltpu.VMEM((AP * NB,), I32),
            pltpu.VMEM((AP * NB,), I32),
            pltpu.VMEM((G * NB, H), F32),
            pltpu.VMEM((G * NB, H), F32),
            pltpu.VMEM((G, H), F32),
            pltpu.VMEM((G, H), F32),
            pltpu.VMEM((G, H), F32),
            pltpu.SemaphoreType.DMA,
            pltpu.SemaphoreType.DMA,
            pltpu.SemaphoreType.DMA,
            pltpu.SemaphoreType.DMA,
            pltpu.SemaphoreType.DMA,
        ],
    )
    def k(atom_hbm, bond_hbm, a2a_hbm, a2b_hbm, outa_hbm, outb_hbm,
          idx_a, idx_b, nbr_a, nbr_b, selfr, outra, outrb,
          sa, sb, ss, swa, swb):
        wid = lax.axis_index("s") * _NC + lax.axis_index("c")
        base = wid * AP

        pltpu.sync_copy(a2a_hbm.at[pl.ds(base * NB, AP * NB)], idx_a)
        pltpu.sync_copy(a2b_hbm.at[pl.ds(base * NB, AP * NB)], idx_b)

        def valid(c):
            return base + c * G < A

        def gathers_a(c):
            return (
                pltpu.make_async_copy(
                    atom_hbm.at[idx_a.at[pl.ds(c * G * NB, G * NB)]],
                    nbr_a, sa),
                pltpu.make_async_copy(
                    atom_hbm.at[pl.ds(base + c * G, G)], selfr, ss),
            )

        def gathers_b(c):
            return (
                pltpu.make_async_copy(
                    bond_hbm.at[idx_b.at[pl.ds(c * G * NB, G * NB)]],
                    nbr_b, sb),
            )

        def issue(cps):
            for cp in cps:
                cp.start()

        @pl.when(valid(0))
        def _():
            issue(gathers_a(0))
            issue(gathers_b(0))

        def body(c, carry):
            @pl.when(valid(c))
            def _():
                for cp in gathers_a(c):
                    cp.wait()

                @pl.when(c > 0)
                def _():
                    pltpu.make_async_copy(
                        outa_hbm.at[pl.ds(0, G)], outra, swa).wait()

                def col_a(cc, carry2):
                    s = pl.ds(cc * 16, 16)
                    for a in range(G):
                        acc = selfr[a, s]
                        for j in range(NB):
                            acc = acc + nbr_a[a * NB + j, s]
                        outra[a, s] = acc
                    return carry2
                lax.fori_loop(0, H // 16, col_a, 0)

                @pl.when(valid(c + 1) & (c + 1 < NCH))
                def _():
                    issue(gathers_a(c + 1))
                pltpu.async_copy(
                    outra, outa_hbm.at[pl.ds(base + c * G, G)], swa)

                for cp in gathers_b(c):
                    cp.wait()

                @pl.when(c > 0)
                def _():
                    pltpu.make_async_copy(
                        outb_hbm.at[pl.ds(0, G)], outrb, swb).wait()

                def col_b(cc, carry2):
                    s = pl.ds(cc * 16, 16)
                    for a in range(G):
                        acc2 = nbr_b[a * NB, s]
                        for j in range(1, NB):
                            acc2 = acc2 + nbr_b[a * NB + j, s]
                        outrb[a, s] = acc2
                    return carry2
                lax.fori_loop(0, H // 16, col_b, 0)

                @pl.when(valid(c + 1) & (c + 1 < NCH))
                def _():
                    issue(gathers_b(c + 1))
                pltpu.async_copy(
                    outrb, outb_hbm.at[pl.ds(base + c * G, G)], swb)
            return carry

        lax.fori_loop(0, NCH, body, 0)

        @pl.when(valid(0))
        def _():
            pltpu.make_async_copy(outa_hbm.at[pl.ds(0, G)], outra, swa).wait()
            pltpu.make_async_copy(outb_hbm.at[pl.ds(0, G)], outrb, swb).wait()

    return k(atom_output, bond_output, a2a_flat, a2b_flat)


def _build_aggr(agg_atom, agg_bond, atom_output, bond_output, b2a, b2revb):
    A, H = agg_atom.shape
    E = b2a.shape[0]
    PER_W = E // _NW
    C = 32
    NCH = PER_W // C
    TAIL = PER_W - NCH * C
    EG = 128
    NEG = (PER_W + EG - 1) // EG
    mesh = plsc.VectorSubcoreMesh(core_axis_name="c", subcore_axis_name="s")

    @functools.partial(
        pl.kernel, mesh=mesh,
        out_type=[jax.ShapeDtypeStruct((E, H), F32),
                  jax.ShapeDtypeStruct((E, H), F32)],
        scratch_types=[
            pltpu.VMEM((PER_W,), I32),
            pltpu.VMEM((PER_W,), I32),
            pltpu.VMEM((PER_W,), I32),
            pltpu.VMEM((C, H), F32),
            pltpu.VMEM((C, H), F32),
            pltpu.VMEM((C, H), F32),
            pltpu.VMEM((C, H), F32),
            pltpu.VMEM((C, H), F32),
            pltpu.VMEM((C, H), F32),
            pltpu.SemaphoreType.DMA,
            pltpu.SemaphoreType.DMA,
            pltpu.SemaphoreType.DMA,
            pltpu.SemaphoreType.DMA,
            pltpu.SemaphoreType.DMA,
            pltpu.SemaphoreType.DMA,
            pltpu.SemaphoreType.DMA,
        ],
    )
    def k(ga_hbm, gb_hbm, atom_hbm, bond_hbm, b2a_hbm, brev_hbm,
          outa_hbm, outb_hbm,
          ba_all, brev_all, reva_all, x0, y0, x1, y1, z0, z1,
          sx0, sy0, sx1, sy1, sw0, sw1, se):
        wid = lax.axis_index("s") * _NC + lax.axis_index("c")
        base = wid * PER_W

        pltpu.sync_copy(b2a_hbm.at[pl.ds(base, PER_W)], ba_all)
        pltpu.sync_copy(brev_hbm.at[pl.ds(base, PER_W)], brev_all)
        BATCH = 10
        for j0 in range(0, NEG, BATCH):
            js = range(j0, min(j0 + BATCH, NEG))
            cps = []
            for j in js:
                n = min(EG, PER_W - j * EG)
                cps.append(pltpu.async_copy(
                    b2a_hbm.at[brev_all.at[pl.ds(j * EG, n)]],
                    reva_all.at[pl.ds(j * EG, n)], se))
            for cp in cps:
                cp.wait()

        def gathers_a(c):
            return (
                pltpu.make_async_copy(
                    ga_hbm.at[ba_all.at[pl.ds(c * C, C)]], x0, sx0),
                pltpu.make_async_copy(
                    atom_hbm.at[reva_all.at[pl.ds(c * C, C)]], y0, sy0),
            )

        def gathers_b(c):
            return (
                pltpu.make_async_copy(
                    gb_hbm.at[ba_all.at[pl.ds(c * C, C)]], x1, sx1),
                pltpu.make_async_copy(
                    bond_hbm.at[brev_all.at[pl.ds(c * C, C)]], y1, sy1),
            )

        def issue(cps):
            for cp in cps:
                cp.start()

        def drain(hbm, dst, sem):
            pltpu.make_async_copy(hbm.at[pl.ds(0, C)], dst, sem).wait()

        def subtract(xr, yr, zr):
            def row(r, carry):
                for cc in range(H // 16):
                    s = pl.ds(cc * 16, 16)
                    zr[r, s] = xr[r, s] - yr[r, s]
                return carry
            lax.fori_loop(0, C, row, 0)

        issue(gathers_a(0))
        issue(gathers_b(0))

        def body(c, carry):
            ca_x, ca_y = gathers_a(c)
            ca_x.wait()
            ca_y.wait()

            @pl.when(c > 0)
            def _():
                drain(outa_hbm, z0, sw0)
            subtract(x0, y0, z0)

            @pl.when(c + 1 < NCH)
            def _():
                issue(gathers_a(c + 1))
            pltpu.async_copy(z0, outa_hbm.at[pl.ds(base + c * C, C)], sw0)

            cb_x, cb_y = gathers_b(c)
            cb_x.wait()
            cb_y.wait()

            @pl.when(c > 0)
            def _():
                drain(outb_hbm, z1, sw1)
            subtract(x1, y1, z1)

            @pl.when(c + 1 < NCH)
            def _():
                issue(gathers_b(c + 1))
            pltpu.async_copy(z1, outb_hbm.at[pl.ds(base + c * C, C)], sw1)
            return carry

        lax.fori_loop(0, NCH, body, 0)

        drain(outa_hbm, z0, sw0)
        drain(outb_hbm, z1, sw1)

        if TAIL:
            t0 = NCH * C
            pltpu.async_copy(
                ga_hbm.at[ba_all.at[pl.ds(t0, TAIL)]],
                x0.at[pl.ds(0, TAIL)], sx0).wait()
            pltpu.async_copy(
                atom_hbm.at[reva_all.at[pl.ds(t0, TAIL)]],
                y0.at[pl.ds(0, TAIL)], sy0).wait()
            pltpu.async_copy(
                gb_hbm.at[ba_all.at[pl.ds(t0, TAIL)]],
                x1.at[pl.ds(0, TAIL)], sx1).wait()
            pltpu.async_copy(
                bond_hbm.at[brev_all.at[pl.ds(t0, TAIL)]],
                y1.at[pl.ds(0, TAIL)], sy1).wait()

            def trow(r, carry):
                def tcol(cc, carry2):
                    s = pl.ds(cc * 16, 16)
                    z0[r, s] = x0[r, s] - y0[r, s]
                    z1[r, s] = x1[r, s] - y1[r, s]
                    return carry2
                return lax.fori_loop(0, H // 16, tcol, carry)
            lax.fori_loop(0, TAIL, trow, 0)
            pltpu.sync_copy(z0.at[pl.ds(0, TAIL)],
                            outa_hbm.at[pl.ds(base + t0, TAIL)])
            pltpu.sync_copy(z1.at[pl.ds(0, TAIL)],
                            outb_hbm.at[pl.ds(base + t0, TAIL)])

    return k(agg_atom, agg_bond, atom_output, bond_output, b2a, b2revb)


def _ffn_body(f_ref, aga_ref, agb_ref,
              w1a_ref, b1a_ref, w2a_ref, b2a_ref, ga_ref, bea_ref,
              w1b_ref, b1b_ref, w2b_ref, b2b_ref, gb_ref, beb_ref,
              oa_ref, ob_ref):
    f = f_ref[...]
    branches = (
        (aga_ref, w1a_ref, b1a_ref, w2a_ref, b2a_ref, ga_ref, bea_ref, oa_ref),
        (agb_ref, w1b_ref, b1b_ref, w2b_ref, b2b_ref, gb_ref, beb_ref, ob_ref),
    )
    for agr, w1r, b1r, w2r, b2r, gr, ber, outr in branches:
        x = jnp.concatenate([f, agr[...]], axis=1).astype(BF16)
        h = jnp.dot(x, w1r[...], preferred_element_type=F32) + b1r[...]
        h = jnp.maximum(h, 0.0).astype(BF16)
        y = jnp.dot(h, w2r[...], preferred_element_type=F32) + b2r[...]
        m = jnp.mean(y, axis=1, keepdims=True)
        yc = y - m
        v = jnp.mean(yc * yc, axis=1, keepdims=True)
        outr[...] = yc * lax.rsqrt(v + 1e-5) * gr[...] + ber[...]


def _run_ffn(f_bonds, aggr_a, aggr_b, W1a, b1a, W2a, b2a_, ga, bea,
             W1b, b1b, W2b, b2b_, gb, beb):
    E, H = f_bonds.shape
    D = W1a.shape[1]
    BE = 640
    grid = (E // BE,)
    row_spec = lambda w: pl.BlockSpec((BE, w), lambda i: (i, 0))
    full_spec = lambda r, c: pl.BlockSpec((r, c), lambda i: (0, 0))
    out = pl.pallas_call(
        _ffn_body,
        grid=grid,
        in_specs=[
            row_spec(H), row_spec(H), row_spec(H),
            full_spec(2 * H, D), full_spec(1, D), full_spec(D, H),
            full_spec(1, H), full_spec(1, H), full_spec(1, H),
            full_spec(2 * H, D), full_spec(1, D), full_spec(D, H),
            full_spec(1, H), full_spec(1, H), full_spec(1, H),
        ],
        out_specs=[row_spec(H), row_spec(H)],
        out_shape=[jax.ShapeDtypeStruct((E, H), F32),
                   jax.ShapeDtypeStruct((E, H), F32)],
    )(f_bonds, aggr_a, aggr_b,
      W1a, b1a, W2a, b2a_, ga, bea,
      W1b, b1b, W2b, b2b_, gb, beb)
    return out


def _readout_body(bia_ref, sizes_ref, feats_ref, w1m_ref, w1f_ref, b1_ref,
                  w2_ref, b2_ref, out_ref, *, n_mols, mol_sz):
    x = bia_ref[...]
    H = x.shape[1]
    xr = x.reshape(n_mols, mol_sz, H)
    sizes = sizes_ref[...]
    offs = lax.broadcasted_iota(I32, (n_mols, mol_sz), 1).astype(F32)
    mask = (offs < sizes).astype(F32)
    mv = (xr * mask[:, :, None]).sum(axis=1) / sizes
    h = jnp.dot(mv, w1m_ref[...], preferred_element_type=F32)
    h = h + jnp.dot(feats_ref[...], w1f_ref[...], preferred_element_type=F32)
    h = jnp.maximum(h + b1_ref[...], 0.0)
    y = jnp.dot(h, w2_ref[...], preferred_element_type=F32) + b2_ref[...]
    out_ref[...] = jax.nn.sigmoid(y) * 0.5


def _run_readout(bond_in_atom_out, sizes_f, features, Wf1, bf1, Wf2, bf2):
    NA, H = bond_in_atom_out.shape
    M, FT = features.shape
    mol_sz = NA // M
    FH = Wf1.shape[1]
    T = Wf2.shape[1]
    body = functools.partial(_readout_body, n_mols=M, mol_sz=mol_sz)
    return pl.pallas_call(
        body,
        out_shape=jax.ShapeDtypeStruct((M, T), F32),
    )(bond_in_atom_out, sizes_f, features,
      Wf1[:H], Wf1[H:], bf1.reshape(1, FH), Wf2, bf2.reshape(1, T))


def kernel(atom_output, bond_output, original_f_atoms, original_f_bonds,
           a2a, a2b, b2a, b2revb, a_scope, features_batch, bond_in_atom_out,
           W1_bfa, b1_bfa, W2_bfa, b2_bfa, g_bfa, be_bfa,
           W1_bfb, b1_bfb, W2_bfb, b2_bfb, g_bfb, be_bfb,
           Wf1, bf1, Wf2, bf2):
    pad_to = _NW * 320 * 16
    a2a_flat = a2a.astype(I32).reshape(-1)
    a2b_flat = a2b.astype(I32).reshape(-1)
    if a2a_flat.shape[0] < pad_to:
        a2a_flat = jnp.pad(a2a_flat, (0, pad_to - a2a_flat.shape[0]))
        a2b_flat = jnp.pad(a2b_flat, (0, pad_to - a2b_flat.shape[0]))
    b2a_i = b2a.astype(I32)
    brev_i = b2revb.astype(I32)

    agg_atom, agg_bond = _build_agg_tables(
        atom_output, bond_output, a2a_flat, a2b_flat)
    aggr_a, aggr_b = _build_aggr(
        agg_atom, agg_bond, atom_output, bond_output, b2a_i, brev_i)

    out_a, out_b = _run_ffn(
        original_f_bonds, aggr_a, aggr_b,
        W1_bfa.astype(BF16), b1_bfa.reshape(1, -1), W2_bfa.astype(BF16),
        b2_bfa.reshape(1, -1), g_bfa.reshape(1, -1), be_bfa.reshape(1, -1),
        W1_bfb.astype(BF16), b1_bfb.reshape(1, -1), W2_bfb.astype(BF16),
        b2_bfb.reshape(1, -1), g_bfb.reshape(1, -1), be_bfb.reshape(1, -1))

    sizes_f = a_scope[:, 1].astype(F32).reshape(-1, 1)
    output = _run_readout(bond_in_atom_out, sizes_f, features_batch,
                          Wf1, bf1, Wf2, bf2)
    return output, out_a, out_b
